# baseline (speedup 1.0000x reference)
import jax
import jax.numpy as jnp
from jax import lax
from jax.experimental import pallas as pl
from jax.experimental.pallas import tpu as pltpu

_SCALING = 2.0
_TN = 512


def _moe_tile_kernel(xf_ref, bnd_ref, xk_ref, rt_ref, kreft_ref, vref_ref,
                     ka_ref, kb_ref, va_ref, vb_ref, out_ref,
                     kreft_s, vref_s, ka_st, kb_st, va_st, vb_st,
                     ka2_s, kb_s, va2_s, vb_s, sems):
    f32 = jnp.float32
    bf16 = jnp.bfloat16

    @pl.when(pl.program_id(0) == 0)
    def _prep():
        copies = [
            pltpu.make_async_copy(kreft_ref, kreft_s, sems.at[0]),
            pltpu.make_async_copy(vref_ref, vref_s, sems.at[1]),
            pltpu.make_async_copy(ka_ref, ka_st, sems.at[2]),
            pltpu.make_async_copy(kb_ref, kb_st, sems.at[3]),
            pltpu.make_async_copy(va_ref, va_st, sems.at[4]),
            pltpu.make_async_copy(vb_ref, vb_st, sems.at[5]),
        ]
        for c in copies:
            c.start()
        for c in copies:
            c.wait()
        e, r, d = ka_st.shape
        ka2_s[...] = (_SCALING * ka_st[...]).reshape(e * r, d).astype(bf16)
        kb_s[...] = kb_st[...].astype(bf16)
        va2_s[...] = (_SCALING * va_st[...]).astype(bf16)
        vb_s[...] = vb_st[...].astype(bf16)

    xf = xf_ref[...]
    tn = xf.shape[0]
    xs = jnp.concatenate([bnd_ref[0], xf[:-1, :]], axis=0)
    hid = xf + (xs - xf) * xk_ref[...]

    scores = lax.dot_general(hid, rt_ref[...], (((1,), (0,)), ((), ())),
                             preferred_element_type=f32)
    e_cnt = scores.shape[1]
    iota = lax.broadcasted_iota(jnp.int32, (tn, e_cnt), 1)
    m1 = jnp.max(scores, axis=1, keepdims=True)
    i1 = jnp.min(jnp.where(scores == m1, iota, e_cnt), axis=1, keepdims=True)
    masked = jnp.where(iota == i1, -jnp.inf, scores)
    m2 = jnp.max(masked, axis=1, keepdims=True)
    i2 = jnp.min(jnp.where(masked == m2, iota, e_cnt), axis=1, keepdims=True)
    w2 = jnp.exp(m2 - m1)
    denom = 1.0 + w2
    g_hi = 1.0 / denom
    g_lo = w2 / denom
    g_all = jnp.where(iota == i1, g_hi, jnp.where(iota == i2, g_lo, 0.0))
    g_t = g_all.astype(bf16).T

    hid_t = hid.astype(bf16).T
    shared_t = lax.dot_general(kreft_s[...], hid_t, (((1,), (0,)), ((), ())),
                               preferred_element_type=f32)
    p_t = lax.dot_general(ka2_s[...], hid_t, (((1,), (0,)), ((), ())),
                          preferred_element_type=f32)

    kbar_t = jnp.zeros(shared_t.shape, bf16)
    lora_v = None
    r_dim = p_t.shape[0] // e_cnt
    for e in range(e_cnt):
        p_e = p_t[e * r_dim:(e + 1) * r_dim, :].astype(bf16)
        lk_t = lax.dot_general(kb_s[e], p_e, (((1,), (0,)), ((), ())),
                               preferred_element_type=f32)
        r_t = jnp.maximum(shared_t + lk_t, 0.0).astype(bf16)
        gk_t = (r_t * r_t) * g_t[e:e + 1, :]
        kbar_t = kbar_t + gk_t
        q_t = lax.dot_general(va2_s[e], gk_t,
                              (((1,), (0,)), ((), ())),
                              preferred_element_type=f32)
        lv = lax.dot_general(q_t.astype(bf16), vb_s[e],
                             (((0,), (1,)), ((), ())),
                             preferred_element_type=f32)
        lora_v = lv if lora_v is None else lora_v + lv

    out = lax.dot_general(kbar_t, vref_s[...],
                          (((0,), (0,)), ((), ())),
                          preferred_element_type=f32)
    out_ref[...] = out + lora_v


def kernel(x, x_prev, x_k, Router_ref, K_ref, V_ref,
           Experts_K_a, Experts_K_b, Experts_V_a, Experts_V_b):
    f32 = jnp.float32
    bf16 = jnp.bfloat16
    B, S, D = x.shape
    F = K_ref.shape[1]
    E, R, _ = Experts_K_a.shape
    N = B * S
    nblk = N // _TN

    xf = x.reshape(N, D)
    xk = x_k.reshape(1, D).astype(f32)
    bnd = jnp.concatenate([x_prev, xf[_TN - 1:N - 1:_TN, :]],
                          axis=0).reshape(nblk, 1, D)

    rt = jnp.concatenate([jnp.zeros((1, D), f32), Router_ref], axis=0).T

    kreft_bf = K_ref.T.astype(bf16)
    vref_bf = V_ref.astype(bf16)

    grid = (nblk,)
    fixed = lambda i: (0, 0)
    hbm = pl.BlockSpec(memory_space=pl.ANY)
    out = pl.pallas_call(
        _moe_tile_kernel,
        grid=grid,
        in_specs=[
            pl.BlockSpec((_TN, D), lambda i: (i, 0)),
            pl.BlockSpec((1, 1, D), lambda i: (i, 0, 0)),
            pl.BlockSpec((1, D), fixed),
            pl.BlockSpec((D, E), fixed),
            hbm, hbm, hbm, hbm, hbm, hbm,
        ],
        out_specs=pl.BlockSpec((_TN, D), lambda i: (i, 0)),
        out_shape=jax.ShapeDtypeStruct((N, D), f32),
        scratch_shapes=[
            pltpu.VMEM((F, D), bf16),
            pltpu.VMEM((F, D), bf16),
            pltpu.VMEM((E, R, D), f32),
            pltpu.VMEM((E, F, R), f32),
            pltpu.VMEM((E, R, F), f32),
            pltpu.VMEM((E, D, R), f32),
            pltpu.VMEM((E * R, D), bf16),
            pltpu.VMEM((E, F, R), bf16),
            pltpu.VMEM((E, R, F), bf16),
            pltpu.VMEM((E, D, R), bf16),
            pltpu.SemaphoreType.DMA((6,)),
        ],
        compiler_params=pltpu.CompilerParams(
            dimension_semantics=("arbitrary",),
        ),
    )(xf, bnd, xk, rt, kreft_bf, vref_bf,
      Experts_K_a, Experts_K_b, Experts_V_a, Experts_V_b)

    return (out.reshape(B, S, D), x[:, -1, :])


# speedup vs baseline: 1.1420x; 1.1420x over previous
import jax
import jax.numpy as jnp
from jax import lax
from jax.experimental import pallas as pl
from jax.experimental.pallas import tpu as pltpu

_SCALING = 2.0
_FC = 512


def _moe_kernel(xf_ref, xp_ref, xk_ref, rt_ref, kreft_ref, vref_ref,
                ka_ref, kb_ref, va_ref, vb_ref, out_ref):
    f32 = jnp.float32
    bf16 = jnp.bfloat16

    xf = xf_ref[...]
    n_tok = xf.shape[0]
    xs = jnp.concatenate([xp_ref[...], xf[:-1, :]], axis=0)
    hid = xf + (xs - xf) * xk_ref[...]

    scores = lax.dot_general(hid, rt_ref[...], (((1,), (0,)), ((), ())),
                             preferred_element_type=f32)
    e_cnt = scores.shape[1]
    iota = lax.broadcasted_iota(jnp.int32, (n_tok, e_cnt), 1)
    m1 = jnp.max(scores, axis=1, keepdims=True)
    i1 = jnp.min(jnp.where(scores == m1, iota, e_cnt), axis=1, keepdims=True)
    masked = jnp.where(iota == i1, -jnp.inf, scores)
    m2 = jnp.max(masked, axis=1, keepdims=True)
    i2 = jnp.min(jnp.where(masked == m2, iota, e_cnt), axis=1, keepdims=True)
    w2 = jnp.exp(m2 - m1)
    denom = 1.0 + w2
    g_hi = 1.0 / denom
    g_lo = w2 / denom
    g_all = jnp.where(iota == i1, g_hi, jnp.where(iota == i2, g_lo, 0.0))
    g_t = g_all.astype(bf16).T

    hid_t = hid.astype(bf16).T
    p_t = lax.dot_general(ka_ref[...], hid_t, (((1,), (0,)), ((), ())),
                          preferred_element_type=f32).astype(bf16)
    r_dim = p_t.shape[0] // e_cnt
    p_es = [p_t[e * r_dim:(e + 1) * r_dim, :] for e in range(e_cnt)]
    g_es = [g_t[e:e + 1, :] for e in range(e_cnt)]

    f_dim = kreft_ref.shape[0]
    out_acc = None
    q_accs = [None] * e_cnt
    for c in range(f_dim // _FC):
        sl = slice(c * _FC, (c + 1) * _FC)
        shared_c = lax.dot_general(kreft_ref[sl, :], hid_t,
                                   (((1,), (0,)), ((), ())),
                                   preferred_element_type=f32)
        kbar_c = jnp.zeros(shared_c.shape, bf16)
        for e in range(e_cnt):
            lk_c = lax.dot_general(kb_ref[e, sl, :], p_es[e],
                                   (((1,), (0,)), ((), ())),
                                   preferred_element_type=f32)
            r_c = jnp.maximum(shared_c + lk_c, 0.0).astype(bf16)
            gk_c = (r_c * r_c) * g_es[e]
            kbar_c = kbar_c + gk_c
            q_c = lax.dot_general(va_ref[e, :, sl], gk_c,
                                  (((1,), (0,)), ((), ())),
                                  preferred_element_type=f32)
            q_accs[e] = q_c if q_accs[e] is None else q_accs[e] + q_c
        o_c = lax.dot_general(kbar_c, vref_ref[sl, :],
                              (((0,), (0,)), ((), ())),
                              preferred_element_type=f32)
        out_acc = o_c if out_acc is None else out_acc + o_c

    for e in range(e_cnt):
        lv = lax.dot_general(q_accs[e].astype(bf16), vb_ref[e],
                             (((0,), (1,)), ((), ())),
                             preferred_element_type=f32)
        out_acc = out_acc + lv
    out_ref[...] = out_acc


def kernel(x, x_prev, x_k, Router_ref, K_ref, V_ref,
           Experts_K_a, Experts_K_b, Experts_V_a, Experts_V_b):
    f32 = jnp.float32
    bf16 = jnp.bfloat16
    B, S, D = x.shape
    F = K_ref.shape[1]
    E, R, _ = Experts_K_a.shape
    N = B * S

    xf = x.reshape(N, D)
    xk = x_k.reshape(1, D).astype(f32)

    rt = jnp.concatenate([jnp.zeros((1, D), f32), Router_ref], axis=0).T

    kreft_bf = K_ref.T.astype(bf16)
    vref_bf = V_ref.astype(bf16)
    ka2 = (_SCALING * Experts_K_a).reshape(E * R, D).astype(bf16)
    kb_bf = Experts_K_b.astype(bf16)
    va2 = (_SCALING * Experts_V_a).astype(bf16)
    vb_bf = Experts_V_b.astype(bf16)

    out = pl.pallas_call(
        _moe_kernel,
        out_shape=jax.ShapeDtypeStruct((N, D), f32),
    )(xf, x_prev, xk, rt, kreft_bf, vref_bf, ka2, kb_bf, va2, vb_bf)

    return (out.reshape(B, S, D), x[:, -1, :])
